# baseline (speedup 1.0000x reference)
import functools

import jax
import jax.numpy as jnp
from jax import lax
from jax.experimental import pallas as pl
from jax.experimental.pallas import tpu as pltpu
from jax.experimental.pallas import tpu_sc as plsc


def _pad128(n):
    return (n + 1023) // 1024 * 1024


def _sc_mesh():
    return plsc.VectorSubcoreMesh(core_axis_name="c", subcore_axis_name="s")


def _make_sc_deg(E, N, NC, NS):
    NW = NC * NS
    EPW = E // NW
    NP = _pad128(N)

    @functools.partial(
        pl.kernel,
        mesh=_sc_mesh(),
        out_type=jax.ShapeDtypeStruct((NW * NP,), jnp.float32),
        scratch_types=[
            pltpu.VMEM((EPW,), jnp.int32),
            pltpu.VMEM((NP,), jnp.float32),
        ],
        compiler_params=pltpu.CompilerParams(needs_layout_passes=False),
    )
    def deg_kernel(dst_hbm, out_hbm, dst_v, acc_v):
        wid = lax.axis_index("s") * NC + lax.axis_index("c")
        pltpu.sync_copy(dst_hbm.at[pl.ds(wid * EPW, EPW)], dst_v)
        zeros = jnp.zeros((16,), jnp.float32)

        def zbody(i, carry):
            acc_v[pl.ds(i * 16, 16)] = zeros
            return carry

        lax.fori_loop(0, NP // 16, zbody, 0, unroll=8)
        ones = jnp.ones((16,), jnp.float32)

        def ebody(i, carry):
            di = dst_v[pl.ds(i * 16, 16)]
            plsc.addupdate_scatter(acc_v, [di], ones)
            return carry

        lax.fori_loop(0, EPW // 16, ebody, 0, unroll=8)
        pltpu.sync_copy(acc_v, out_hbm.at[pl.ds(wid * NP, NP)])

    return deg_kernel


def _make_sc_spmm(E, N, NC, NS):
    NW = NC * NS
    EPW = E // NW
    NP = _pad128(N)

    @functools.partial(
        pl.kernel,
        mesh=_sc_mesh(),
        out_type=jax.ShapeDtypeStruct((NW * NP,), jnp.float32),
        scratch_types=[
            pltpu.VMEM((EPW,), jnp.int32),
            pltpu.VMEM((EPW,), jnp.int32),
            pltpu.VMEM((N,), jnp.float32),
            pltpu.VMEM((NP,), jnp.float32),
        ],
        compiler_params=pltpu.CompilerParams(needs_layout_passes=False),
    )
    def spmm_kernel(srcdst_hbm, v_hbm, out_hbm, src_v, dst_v, v_v, acc_v):
        wid = lax.axis_index("s") * NC + lax.axis_index("c")
        pltpu.sync_copy(srcdst_hbm.at[pl.ds(wid * EPW, EPW)], src_v)
        pltpu.sync_copy(srcdst_hbm.at[pl.ds(E + wid * EPW, EPW)], dst_v)
        pltpu.sync_copy(v_hbm, v_v)
        zeros = jnp.zeros((16,), jnp.float32)

        def zbody(i, carry):
            acc_v[pl.ds(i * 16, 16)] = zeros
            return carry

        lax.fori_loop(0, NP // 16, zbody, 0, unroll=8)

        def ebody(i, carry):
            si = src_v[pl.ds(i * 16, 16)]
            di = dst_v[pl.ds(i * 16, 16)]
            vals = plsc.load_gather(v_v, [si])
            plsc.addupdate_scatter(acc_v, [di], vals)
            return carry

        lax.fori_loop(0, EPW // 16, ebody, 0, unroll=16)
        pltpu.sync_copy(acc_v, out_hbm.at[pl.ds(wid * NP, NP)])

    return spmm_kernel


def _tc_matvec_u2(x, s, W2):
    N, D = x.shape

    def body(x_ref, s_ref, w_ref, u2_ref):
        ws = s_ref[...][None, :] * w_ref[...]
        u = lax.dot_general(
            x_ref[...], ws, (((1,), (1,)), ((), ())),
            preferred_element_type=jnp.float32,
            precision=lax.Precision.DEFAULT,
        )
        u2_ref[...] = u[:, 0]

    return pl.pallas_call(
        body,
        out_shape=jax.ShapeDtypeStruct((N,), jnp.float32),
    )(x, s, W2)


def _tc_matvec_u01(x, s, W01, after):
    N, D = x.shape

    def body(x_ref, s_ref, w_ref, a_ref, u0_ref, u1_ref):
        ws = s_ref[...][None, :] * w_ref[...]
        u = lax.dot_general(
            x_ref[...], ws, (((1,), (1,)), ((), ())),
            preferred_element_type=jnp.float32,
            precision=lax.Precision.DEFAULT,
        )
        u0_ref[...] = u[:, 0]
        u1_ref[...] = u[:, 1]

    f32 = jnp.float32
    return pl.pallas_call(
        body,
        out_shape=[jax.ShapeDtypeStruct((N,), f32)] * 2,
    )(x, s, W01, after)


def _reduce_partials_call(body_last, partials, extra_inputs, n_out, N, NW, NP):
    def body(*refs):
        p_ref = refs[0]
        extras = refs[1:1 + len(extra_inputs)]
        outs = refs[1 + len(extra_inputs):]
        total = p_ref[pl.ds(0, NP)]
        for k in range(1, NW):
            total += p_ref[pl.ds(k * NP, NP)]
        body_last(total[:N], extras, outs)

    f32 = jnp.float32
    return pl.pallas_call(
        body,
        out_shape=[jax.ShapeDtypeStruct((N,), f32)] * n_out,
    )(partials, *extra_inputs)


def _tc_dinv(degp, u2, N, NW, NP):
    def last(deg, extras, outs):
        u2_ref, = extras
        dinv_ref, q2_ref = outs
        dinv = jnp.where(deg > 0, lax.rsqrt(jnp.maximum(deg, 1e-12)), 0.0)
        dinv_ref[...] = dinv
        q2_ref[...] = dinv * u2_ref[...]

    return _reduce_partials_call(last, degp, [u2], 2, N, NW, NP)


def _tc_mid(y1p, u1, dinv, N, NW, NP):
    def last(y1, extras, outs):
        u1_ref, dinv_ref = extras
        g_ref, = outs
        dinv = dinv_ref[...]
        g_ref[...] = dinv * (u1_ref[...] + dinv * y1)

    return _reduce_partials_call(last, y1p, [u1, dinv], 1, N, NW, NP)


def _tc_post(y2p, u0, dinv, b, N, NW, NP):
    def body(p_ref, u0_ref, dinv_ref, b_ref, out_ref):
        total = p_ref[pl.ds(0, NP)]
        for k in range(1, NW):
            total += p_ref[pl.ds(k * NP, NP)]
        val = u0_ref[...] + dinv_ref[...] * total[:N] + b_ref[...]
        out_ref[...] = val[:, None]

    return pl.pallas_call(
        body,
        out_shape=jax.ShapeDtypeStruct((N, 1), jnp.float32),
    )(y2p, u0, dinv, b)


def kernel(x, edge_index, s, W, b):
    N, D = x.shape
    E = edge_index.shape[1]
    K = W.shape[0] // D

    info = plsc.get_sparse_core_info()
    NC, NS = info.num_cores, info.num_subcores
    NW = NC * NS
    NP = _pad128(N)

    dstarr = edge_index[1]
    srcdst = edge_index.reshape(2 * E)
    Wr = W[:, 0].reshape(K, D)

    deg_k = _make_sc_deg(E, N, NC, NS)
    spmm_k = _make_sc_spmm(E, N, NC, NS)

    degp = deg_k(dstarr)
    u2 = _tc_matvec_u2(x, s, Wr[2:3])
    dinv, q2 = _tc_dinv(degp, u2, N, NW, NP)
    y1p = spmm_k(srcdst, q2)
    u0, u1 = _tc_matvec_u01(x, s, Wr[0:2], dinv)
    g = _tc_mid(y1p, u1, dinv, N, NW, NP)[0]
    y2p = spmm_k(srcdst, g)
    out = _tc_post(y2p, u0, dinv, b, N, NW, NP)
    return out


# speedup vs baseline: 1.0019x; 1.0019x over previous
import functools

import jax
import jax.numpy as jnp
from jax import lax
from jax.experimental import pallas as pl
from jax.experimental.pallas import tpu as pltpu
from jax.experimental.pallas import tpu_sc as plsc


def _pad128(n):
    return (n + 1023) // 1024 * 1024


def _sc_mesh():
    return plsc.VectorSubcoreMesh(core_axis_name="c", subcore_axis_name="s")


def _make_sc_deg(E, N, NC, NS):
    NW = NC * NS
    EPW = E // NW
    NP = _pad128(N)

    @functools.partial(
        pl.kernel,
        mesh=_sc_mesh(),
        out_type=jax.ShapeDtypeStruct((NW * NP,), jnp.float32),
        scratch_types=[
            pltpu.VMEM((EPW,), jnp.int32),
            pltpu.VMEM((NP,), jnp.float32),
        ],
        compiler_params=pltpu.CompilerParams(needs_layout_passes=False),
    )
    def deg_kernel(dst_hbm, zeros_hbm, out_hbm, dst_v, acc_v):
        wid = lax.axis_index("s") * NC + lax.axis_index("c")
        pltpu.sync_copy(dst_hbm.at[pl.ds(wid * EPW, EPW)], dst_v)
        pltpu.sync_copy(zeros_hbm, acc_v)
        ones = jnp.ones((16,), jnp.float32)

        def ebody(i, carry):
            di = dst_v[pl.ds(i * 16, 16)]
            plsc.addupdate_scatter(acc_v, [di], ones)
            return carry

        lax.fori_loop(0, EPW // 16, ebody, 0, unroll=8)
        pltpu.sync_copy(acc_v, out_hbm.at[pl.ds(wid * NP, NP)])

    return deg_kernel


def _make_sc_spmm(E, N, NC, NS):
    NW = NC * NS
    EPW = E // NW
    NP = _pad128(N)

    @functools.partial(
        pl.kernel,
        mesh=_sc_mesh(),
        out_type=jax.ShapeDtypeStruct((NW * NP,), jnp.float32),
        scratch_types=[
            pltpu.VMEM((EPW,), jnp.int32),
            pltpu.VMEM((EPW,), jnp.int32),
            pltpu.VMEM((N,), jnp.float32),
            pltpu.VMEM((NP,), jnp.float32),
        ],
        compiler_params=pltpu.CompilerParams(needs_layout_passes=False),
    )
    def spmm_kernel(src_hbm, dst_hbm, v_hbm, zeros_hbm, out_hbm, src_v, dst_v, v_v, acc_v):
        wid = lax.axis_index("s") * NC + lax.axis_index("c")
        pltpu.sync_copy(src_hbm.at[pl.ds(wid * EPW, EPW)], src_v)
        pltpu.sync_copy(dst_hbm.at[pl.ds(wid * EPW, EPW)], dst_v)
        pltpu.sync_copy(v_hbm, v_v)
        pltpu.sync_copy(zeros_hbm, acc_v)

        def ebody(i, carry):
            si = src_v[pl.ds(i * 16, 16)]
            di = dst_v[pl.ds(i * 16, 16)]
            vals = plsc.load_gather(v_v, [si])
            plsc.addupdate_scatter(acc_v, [di], vals)
            return carry

        lax.fori_loop(0, EPW // 16, ebody, 0, unroll=8)
        pltpu.sync_copy(acc_v, out_hbm.at[pl.ds(wid * NP, NP)])

    return spmm_kernel


def _tc_matvec_u2(x, s, W2):
    N, D = x.shape

    def body(x_ref, s_ref, w_ref, u2_ref):
        ws = s_ref[...][None, :] * w_ref[...]
        u = lax.dot_general(
            x_ref[...], ws, (((1,), (1,)), ((), ())),
            preferred_element_type=jnp.float32,
            precision=lax.Precision.DEFAULT,
        )
        u2_ref[...] = u[:, 0]

    return pl.pallas_call(
        body,
        out_shape=jax.ShapeDtypeStruct((N,), jnp.float32),
    )(x, s, W2)


def _tc_matvec_u01(x, s, W01, after):
    N, D = x.shape

    def body(x_ref, s_ref, w_ref, a_ref, u0_ref, u1_ref):
        ws = s_ref[...][None, :] * w_ref[...]
        u = lax.dot_general(
            x_ref[...], ws, (((1,), (1,)), ((), ())),
            preferred_element_type=jnp.float32,
            precision=lax.Precision.DEFAULT,
        )
        u0_ref[...] = u[:, 0]
        u1_ref[...] = u[:, 1]

    f32 = jnp.float32
    return pl.pallas_call(
        body,
        out_shape=[jax.ShapeDtypeStruct((N,), f32)] * 2,
    )(x, s, W01, after)


def _reduce_partials_call(body_last, partials, extra_inputs, n_out, N, NW, NP):
    def body(*refs):
        p_ref = refs[0]
        extras = refs[1:1 + len(extra_inputs)]
        outs = refs[1 + len(extra_inputs):]
        total = p_ref[pl.ds(0, NP)]
        for k in range(1, NW):
            total += p_ref[pl.ds(k * NP, NP)]
        body_last(total[:N], extras, outs)

    f32 = jnp.float32
    return pl.pallas_call(
        body,
        out_shape=[jax.ShapeDtypeStruct((N,), f32)] * n_out,
    )(partials, *extra_inputs)


def _tc_dinv(degp, u2, N, NW, NP):
    def last(deg, extras, outs):
        u2_ref, = extras
        dinv_ref, q2_ref = outs
        dinv = jnp.where(deg > 0, lax.rsqrt(jnp.maximum(deg, 1e-12)), 0.0)
        dinv_ref[...] = dinv
        q2_ref[...] = dinv * u2_ref[...]

    return _reduce_partials_call(last, degp, [u2], 2, N, NW, NP)


def _tc_mid(y1p, u1, dinv, N, NW, NP):
    def last(y1, extras, outs):
        u1_ref, dinv_ref = extras
        g_ref, = outs
        dinv = dinv_ref[...]
        g_ref[...] = dinv * (u1_ref[...] + dinv * y1)

    return _reduce_partials_call(last, y1p, [u1, dinv], 1, N, NW, NP)


def _tc_post(y2p, u0, dinv, b, N, NW, NP):
    def body(p_ref, u0_ref, dinv_ref, b_ref, out_ref):
        total = p_ref[pl.ds(0, NP)]
        for k in range(1, NW):
            total += p_ref[pl.ds(k * NP, NP)]
        val = u0_ref[...] + dinv_ref[...] * total[:N] + b_ref[...]
        out_ref[...] = val[:, None]

    return pl.pallas_call(
        body,
        out_shape=jax.ShapeDtypeStruct((N, 1), jnp.float32),
    )(y2p, u0, dinv, b)


def kernel(x, edge_index, s, W, b):
    N, D = x.shape
    E = edge_index.shape[1]
    K = W.shape[0] // D

    info = plsc.get_sparse_core_info()
    NC, NS = info.num_cores, info.num_subcores
    NW = NC * NS
    NP = _pad128(N)

    dstarr = edge_index[1]
    srcarr = edge_index[0]
    zerosv = jnp.zeros((NP,), jnp.float32)
    Wr = W[:, 0].reshape(K, D)

    deg_k = _make_sc_deg(E, N, NC, NS)
    spmm_k = _make_sc_spmm(E, N, NC, NS)

    degp = deg_k(dstarr, zerosv)
    u2 = _tc_matvec_u2(x, s, Wr[2:3])
    dinv, q2 = _tc_dinv(degp, u2, N, NW, NP)
    y1p = spmm_k(srcarr, dstarr, q2, zerosv)
    u0, u1 = _tc_matvec_u01(x, s, Wr[0:2], dinv)
    g = _tc_mid(y1p, u1, dinv, N, NW, NP)[0]
    y2p = spmm_k(srcarr, dstarr, g, zerosv)
    out = _tc_post(y2p, u0, dinv, b, N, NW, NP)
    return out


# speedup vs baseline: 1.1467x; 1.1445x over previous
import functools

import jax
import jax.numpy as jnp
from jax import lax
from jax.experimental import pallas as pl
from jax.experimental.pallas import tpu as pltpu
from jax.experimental.pallas import tpu_sc as plsc


def _pad128(n):
    return (n + 1023) // 1024 * 1024


def _sc_mesh():
    return plsc.VectorSubcoreMesh(core_axis_name="c", subcore_axis_name="s")


def _make_sc_deg(E, N, NC, NS):
    NW = NC * NS
    EPW = E // NW
    NP = _pad128(N)

    @functools.partial(
        pl.kernel,
        mesh=_sc_mesh(),
        out_type=jax.ShapeDtypeStruct((NW * NP,), jnp.float32),
        scratch_types=[
            pltpu.VMEM((EPW,), jnp.int32),
            pltpu.VMEM((NP,), jnp.float32),
            pltpu.SemaphoreType.DMA,
        ],
        compiler_params=pltpu.CompilerParams(needs_layout_passes=False),
    )
    def deg_kernel(srcdst_hbm, zeros_hbm, out_hbm, dst_v, acc_v, sem):
        wid = lax.axis_index("s") * NC + lax.axis_index("c")
        c1 = pltpu.async_copy(srcdst_hbm.at[pl.ds(E + wid * EPW, EPW)], dst_v, sem)
        c2 = pltpu.async_copy(zeros_hbm, acc_v, sem)
        c1.wait()
        c2.wait()
        ones = jnp.ones((16,), jnp.float32)

        def ebody(i, carry):
            di = dst_v[pl.ds(i * 16, 16)]
            plsc.addupdate_scatter(acc_v, [di], ones)
            return carry

        lax.fori_loop(0, EPW // 16, ebody, 0, unroll=8)
        pltpu.sync_copy(acc_v, out_hbm.at[pl.ds(wid * NP, NP)])

    return deg_kernel


def _make_sc_spmm(E, N, NC, NS):
    NW = NC * NS
    EPW = E // NW
    NP = _pad128(N)

    @functools.partial(
        pl.kernel,
        mesh=_sc_mesh(),
        out_type=jax.ShapeDtypeStruct((NW * NP,), jnp.float32),
        scratch_types=[
            pltpu.VMEM((EPW,), jnp.int32),
            pltpu.VMEM((EPW,), jnp.int32),
            pltpu.VMEM((N,), jnp.float32),
            pltpu.VMEM((NP,), jnp.float32),
            pltpu.SemaphoreType.DMA,
        ],
        compiler_params=pltpu.CompilerParams(needs_layout_passes=False),
    )
    def spmm_kernel(srcdst_hbm, v_hbm, zeros_hbm, out_hbm, src_v, dst_v, v_v, acc_v, sem):
        wid = lax.axis_index("s") * NC + lax.axis_index("c")
        c1 = pltpu.async_copy(srcdst_hbm.at[pl.ds(wid * EPW, EPW)], src_v, sem)
        c2 = pltpu.async_copy(srcdst_hbm.at[pl.ds(E + wid * EPW, EPW)], dst_v, sem)
        c3 = pltpu.async_copy(v_hbm, v_v, sem)
        c4 = pltpu.async_copy(zeros_hbm, acc_v, sem)
        c1.wait()
        c2.wait()
        c3.wait()
        c4.wait()

        def ebody(i, carry):
            si = src_v[pl.ds(i * 16, 16)]
            di = dst_v[pl.ds(i * 16, 16)]
            vals = plsc.load_gather(v_v, [si])
            plsc.addupdate_scatter(acc_v, [di], vals)
            return carry

        lax.fori_loop(0, EPW // 16, ebody, 0, unroll=8)
        pltpu.sync_copy(acc_v, out_hbm.at[pl.ds(wid * NP, NP)])

    return spmm_kernel


def _tc_matvec_u2(x, s, W2):
    N, D = x.shape

    def body(x_ref, s_ref, w_ref, u2_ref):
        ws = s_ref[...][None, :] * w_ref[...]
        u = lax.dot_general(
            x_ref[...], ws, (((1,), (1,)), ((), ())),
            preferred_element_type=jnp.float32,
            precision=lax.Precision.DEFAULT,
        )
        u2_ref[...] = u[:, 0]

    return pl.pallas_call(
        body,
        out_shape=jax.ShapeDtypeStruct((N,), jnp.float32),
    )(x, s, W2)


def _tc_matvec_u01(x, s, W01, after):
    N, D = x.shape

    def body(x_ref, s_ref, w_ref, a_ref, u0_ref, u1_ref):
        ws = s_ref[...][None, :] * w_ref[...]
        u = lax.dot_general(
            x_ref[...], ws, (((1,), (1,)), ((), ())),
            preferred_element_type=jnp.float32,
            precision=lax.Precision.DEFAULT,
        )
        u0_ref[...] = u[:, 0]
        u1_ref[...] = u[:, 1]

    f32 = jnp.float32
    return pl.pallas_call(
        body,
        out_shape=[jax.ShapeDtypeStruct((N,), f32)] * 2,
    )(x, s, W01, after)


def _reduce_partials_call(body_last, partials, extra_inputs, n_out, N, NW, NP):
    def body(*refs):
        p_ref = refs[0]
        extras = refs[1:1 + len(extra_inputs)]
        outs = refs[1 + len(extra_inputs):]
        total = p_ref[pl.ds(0, NP)]
        for k in range(1, NW):
            total += p_ref[pl.ds(k * NP, NP)]
        body_last(total[:N], extras, outs)

    f32 = jnp.float32
    return pl.pallas_call(
        body,
        out_shape=[jax.ShapeDtypeStruct((N,), f32)] * n_out,
    )(partials, *extra_inputs)


def _tc_dinv(degp, u2, N, NW, NP):
    def last(deg, extras, outs):
        u2_ref, = extras
        dinv_ref, q2_ref = outs
        dinv = jnp.where(deg > 0, lax.rsqrt(jnp.maximum(deg, 1e-12)), 0.0)
        dinv_ref[...] = dinv
        q2_ref[...] = dinv * u2_ref[...]

    return _reduce_partials_call(last, degp, [u2], 2, N, NW, NP)


def _tc_mid(y1p, u1, dinv, N, NW, NP):
    def last(y1, extras, outs):
        u1_ref, dinv_ref = extras
        g_ref, = outs
        dinv = dinv_ref[...]
        g_ref[...] = dinv * (u1_ref[...] + dinv * y1)

    return _reduce_partials_call(last, y1p, [u1, dinv], 1, N, NW, NP)


def _tc_post(y2p, u0, dinv, b, N, NW, NP):
    def body(p_ref, u0_ref, dinv_ref, b_ref, out_ref):
        total = p_ref[pl.ds(0, NP)]
        for k in range(1, NW):
            total += p_ref[pl.ds(k * NP, NP)]
        val = u0_ref[...] + dinv_ref[...] * total[:N] + b_ref[...]
        out_ref[...] = val[:, None]

    return pl.pallas_call(
        body,
        out_shape=jax.ShapeDtypeStruct((N, 1), jnp.float32),
    )(y2p, u0, dinv, b)


def kernel(x, edge_index, s, W, b):
    N, D = x.shape
    E = edge_index.shape[1]
    K = W.shape[0] // D

    info = plsc.get_sparse_core_info()
    NC, NS = info.num_cores, info.num_subcores
    NW = NC * NS
    NP = _pad128(N)

    srcdst = edge_index.reshape(2 * E)
    zerosv = jnp.zeros((NP,), jnp.float32)
    Wr = W[:, 0].reshape(K, D)

    deg_k = _make_sc_deg(E, N, NC, NS)
    spmm_k = _make_sc_spmm(E, N, NC, NS)

    degp = deg_k(srcdst, zerosv)
    u2 = _tc_matvec_u2(x, s, Wr[2:3])
    dinv, q2 = _tc_dinv(degp, u2, N, NW, NP)
    y1p = spmm_k(srcdst, q2, zerosv)
    u0, u1 = _tc_matvec_u01(x, s, Wr[0:2], dinv)
    g = _tc_mid(y1p, u1, dinv, N, NW, NP)[0]
    y2p = spmm_k(srcdst, g, zerosv)
    out = _tc_post(y2p, u0, dinv, b, N, NW, NP)
    return out


# speedup vs baseline: 1.3664x; 1.1916x over previous
import functools

import jax
import jax.numpy as jnp
from jax import lax
from jax.experimental import pallas as pl
from jax.experimental.pallas import tpu as pltpu
from jax.experimental.pallas import tpu_sc as plsc


def _pad128(n):
    return (n + 1023) // 1024 * 1024


def _sc_mesh():
    return plsc.VectorSubcoreMesh(core_axis_name="c", subcore_axis_name="s")


def _make_sc_deg(E, N, NC, NS):
    NW = NC * NS
    EPW = E // NW
    NP = _pad128(N)

    @functools.partial(
        pl.kernel,
        mesh=_sc_mesh(),
        out_type=jax.ShapeDtypeStruct((NW * NP,), jnp.float32),
        scratch_types=[
            pltpu.VMEM((EPW,), jnp.int32),
            pltpu.VMEM((NP,), jnp.float32),
            pltpu.SemaphoreType.DMA,
        ],
        compiler_params=pltpu.CompilerParams(needs_layout_passes=False),
    )
    def deg_kernel(srcdst_hbm, out_hbm, dst_v, acc_v, sem):
        wid = lax.axis_index("s") * NC + lax.axis_index("c")
        c1 = pltpu.async_copy(srcdst_hbm.at[pl.ds(E + wid * EPW, EPW)], dst_v, sem)
        zeros = jnp.zeros((16,), jnp.float32)

        def zbody(i, carry):
            acc_v[pl.ds(i * 16, 16)] = zeros
            return carry

        lax.fori_loop(0, NP // 16, zbody, 0, unroll=8)
        c1.wait()
        ones = jnp.ones((16,), jnp.float32)

        def ebody(i, carry):
            di = dst_v[pl.ds(i * 16, 16)]
            plsc.addupdate_scatter(acc_v, [di], ones)
            return carry

        lax.fori_loop(0, EPW // 16, ebody, 0, unroll=8)
        pltpu.sync_copy(acc_v, out_hbm.at[pl.ds(wid * NP, NP)])

    return deg_kernel


def _make_sc_spmm(E, N, NC, NS):
    NW = NC * NS
    EPW = E // NW
    NP = _pad128(N)

    @functools.partial(
        pl.kernel,
        mesh=_sc_mesh(),
        out_type=jax.ShapeDtypeStruct((NW * NP,), jnp.float32),
        scratch_types=[
            pltpu.VMEM((EPW,), jnp.int32),
            pltpu.VMEM((EPW,), jnp.int32),
            pltpu.VMEM((N,), jnp.float32),
            pltpu.VMEM((NP,), jnp.float32),
            pltpu.SemaphoreType.DMA,
        ],
        compiler_params=pltpu.CompilerParams(needs_layout_passes=False),
    )
    def spmm_kernel(srcdst_hbm, v_hbm, out_hbm, src_v, dst_v, v_v, acc_v, sem):
        wid = lax.axis_index("s") * NC + lax.axis_index("c")
        c1 = pltpu.async_copy(srcdst_hbm.at[pl.ds(wid * EPW, EPW)], src_v, sem)
        c2 = pltpu.async_copy(srcdst_hbm.at[pl.ds(E + wid * EPW, EPW)], dst_v, sem)
        c3 = pltpu.async_copy(v_hbm, v_v, sem)
        zeros = jnp.zeros((16,), jnp.float32)

        def zbody(i, carry):
            acc_v[pl.ds(i * 16, 16)] = zeros
            return carry

        lax.fori_loop(0, NP // 16, zbody, 0, unroll=8)
        c1.wait()
        c2.wait()
        c3.wait()

        def ebody(i, carry):
            si = src_v[pl.ds(i * 16, 16)]
            di = dst_v[pl.ds(i * 16, 16)]
            vals = plsc.load_gather(v_v, [si])
            plsc.addupdate_scatter(acc_v, [di], vals)
            return carry

        lax.fori_loop(0, EPW // 16, ebody, 0, unroll=8)
        pltpu.sync_copy(acc_v, out_hbm.at[pl.ds(wid * NP, NP)])

    return spmm_kernel


def _tc_matvec_u2(x, s, W2):
    N, D = x.shape

    def body(x_ref, s_ref, w_ref, u2_ref):
        ws = s_ref[...][None, :] * w_ref[...]
        u = lax.dot_general(
            x_ref[...], ws, (((1,), (1,)), ((), ())),
            preferred_element_type=jnp.float32,
            precision=lax.Precision.DEFAULT,
        )
        u2_ref[...] = u[:, 0]

    return pl.pallas_call(
        body,
        out_shape=jax.ShapeDtypeStruct((N,), jnp.float32),
    )(x, s, W2)


def _tc_matvec_u01(x, s, W01, after):
    N, D = x.shape

    def body(x_ref, s_ref, w_ref, a_ref, u0_ref, u1_ref):
        ws = s_ref[...][None, :] * w_ref[...]
        u = lax.dot_general(
            x_ref[...], ws, (((1,), (1,)), ((), ())),
            preferred_element_type=jnp.float32,
            precision=lax.Precision.DEFAULT,
        )
        u0_ref[...] = u[:, 0]
        u1_ref[...] = u[:, 1]

    f32 = jnp.float32
    return pl.pallas_call(
        body,
        out_shape=[jax.ShapeDtypeStruct((N,), f32)] * 2,
    )(x, s, W01, after)


def _reduce_partials_call(body_last, partials, extra_inputs, n_out, N, NW, NP):
    def body(*refs):
        p_ref = refs[0]
        extras = refs[1:1 + len(extra_inputs)]
        outs = refs[1 + len(extra_inputs):]
        total = p_ref[pl.ds(0, NP)]
        for k in range(1, NW):
            total += p_ref[pl.ds(k * NP, NP)]
        body_last(total[:N], extras, outs)

    f32 = jnp.float32
    return pl.pallas_call(
        body,
        out_shape=[jax.ShapeDtypeStruct((N,), f32)] * n_out,
    )(partials, *extra_inputs)


def _tc_dinv(degp, u2, N, NW, NP):
    def last(deg, extras, outs):
        u2_ref, = extras
        dinv_ref, q2_ref = outs
        dinv = jnp.where(deg > 0, lax.rsqrt(jnp.maximum(deg, 1e-12)), 0.0)
        dinv_ref[...] = dinv
        q2_ref[...] = dinv * u2_ref[...]

    return _reduce_partials_call(last, degp, [u2], 2, N, NW, NP)


def _tc_mid(y1p, u1, dinv, N, NW, NP):
    def last(y1, extras, outs):
        u1_ref, dinv_ref = extras
        g_ref, = outs
        dinv = dinv_ref[...]
        g_ref[...] = dinv * (u1_ref[...] + dinv * y1)

    return _reduce_partials_call(last, y1p, [u1, dinv], 1, N, NW, NP)


def _tc_post(y2p, u0, dinv, b, N, NW, NP):
    def body(p_ref, u0_ref, dinv_ref, b_ref, out_ref):
        total = p_ref[pl.ds(0, NP)]
        for k in range(1, NW):
            total += p_ref[pl.ds(k * NP, NP)]
        out_ref[...] = u0_ref[...] + dinv_ref[...] * total[:N] + b_ref[...]

    return pl.pallas_call(
        body,
        out_shape=jax.ShapeDtypeStruct((N,), jnp.float32),
    )(y2p, u0, dinv, b)


def kernel(x, edge_index, s, W, b):
    N, D = x.shape
    E = edge_index.shape[1]
    K = W.shape[0] // D

    info = plsc.get_sparse_core_info()
    NC, NS = info.num_cores, info.num_subcores
    NW = NC * NS
    NP = _pad128(N)

    srcdst = edge_index.reshape(2 * E)
    Wr = W[:, 0].reshape(K, D)

    deg_k = _make_sc_deg(E, N, NC, NS)
    spmm_k = _make_sc_spmm(E, N, NC, NS)

    degp = deg_k(srcdst)
    u2 = _tc_matvec_u2(x, s, Wr[2:3])
    dinv, q2 = _tc_dinv(degp, u2, N, NW, NP)
    y1p = spmm_k(srcdst, q2)
    u0, u1 = _tc_matvec_u01(x, s, Wr[0:2], dinv)
    g = _tc_mid(y1p, u1, dinv, N, NW, NP)[0]
    y2p = spmm_k(srcdst, g)
    out = _tc_post(y2p, u0, dinv, b, N, NW, NP)
    return out.reshape(N, 1)
